# baseline (speedup 1.0000x reference)
import functools

import jax
import jax.numpy as jnp
from jax import lax
from jax.experimental import pallas as pl
from jax.experimental.pallas import tpu as pltpu
from jax.experimental.pallas import tpu_sc as plsc

N_NODES = 10000
N_EDGES = 20000
N_PAIRS = 160000
D = 512

NC = 2
NS = 16
L = 16

NW = NC * NS
PPT = N_PAIRS // NS
CH = 2000
NCH = PPT // CH
CHV = CH // L
BATCH = 64
SELCAP = ((PPT + BATCH - 1) // BATCH) * BATCH + L
NBMAX = SELCAP // BATCH + 1
SHIFT = BATCH.bit_length() - 1

B_CSC = 1536
B_PASSES = 7
E_PAD = B_PASSES * NC * B_CSC
C_CSC = 2176
C_PASSES = 3
V_PAD = C_PASSES * NC * C_CSC


def _seg_agg_body(csc, passes, with_counts,
                  table_hbm, g_hbm, s_hbm, *rest):
    if with_counts:
        (sums_hbm, cnts_hbm, gvb, svb, gsel, ssel3, rows, hist,
         accum, sem, sem2) = rest
    else:
        (sums_hbm, gvb, svb, gsel, ssel3, rows, accum, sem, sem2) = rest
        cnts_hbm = hist = None

    c = lax.axis_index("c")
    t = lax.axis_index("s")
    rpt = csc // NS

    if with_counts:
        zv = jnp.zeros((L,), jnp.float32)
        def hzero(i, _):
            hist[pl.ds(i * L, L)] = zv
            return 0
        lax.fori_loop(0, E_PAD // L, hzero, 0)
        onev = jnp.full((L,), 1.0, jnp.float32)
        @pl.when(c == 0)
        def _():
            for ch in range(NCH):
                pltpu.sync_copy(s_hbm.at[pl.ds(t * PPT + ch * CH, CH)], svb)
                def hbody(i, _):
                    plsc.addupdate_scatter(hist, [svb[pl.ds(i * L, L)]], onev)
                    return 0
                lax.fori_loop(0, CHV, hbody, 0)
        wid = t * NC + c
        pltpu.sync_copy(hist, cnts_hbm.at[pl.ds(wid * E_PAD, E_PAD)])

    for p in range(passes):
        base = (p * NC + c) * csc

        z = jnp.zeros((L,), jnp.float32)
        def zrow(r, _):
            for k in range(D // L):
                rows[r, pl.ds(k * L, L)] = z
            return 0
        lax.fori_loop(0, BATCH, zrow, 0)
        done = 0
        while done < rpt:
            n = min(BATCH, rpt - done)
            pltpu.sync_copy(rows.at[pl.ds(0, n)],
                            accum.at[pl.ds(t * rpt + done, n)])
            done += n
        @pl.when(t == 0)
        def _():
            pltpu.sync_copy(rows.at[pl.ds(0, 8)], accum.at[pl.ds(csc, 8)])
        plsc.subcore_barrier()

        zi = jnp.zeros((L,), jnp.int32)
        def gfill(i, _):
            gsel[pl.ds(i * L, L)] = zi
            return 0
        lax.fori_loop(0, SELCAP // L, gfill, 0)
        dmp = jnp.full((L,), csc, jnp.int32)
        def sfill(j, _):
            for k in range(BATCH // L):
                ssel3[j, pl.ds(k * L, L)] = dmp
            return 0
        lax.fori_loop(0, NBMAX, sfill, 0)

        bvec = jnp.full((L,), base, jnp.int32)
        cvec = jnp.full((L,), csc, jnp.int32)
        def cbody(i, off):
            sv = svb[pl.ds(i * L, L)]
            gv = gvb[pl.ds(i * L, L)]
            lv = sv - bvec
            mask = (lv >= 0) & (lv < cvec)
            mi = mask.astype(jnp.int32)
            cs = plsc.cumsum(mi)
            pos = jnp.full((L,), off, jnp.int32) + cs - mi
            plsc.store_scatter(gsel, [pos], gv, mask=mask)
            plsc.store_scatter(
                ssel3,
                [lax.shift_right_logical(pos, SHIFT), pos & (BATCH - 1)],
                lv, mask=mask)
            return off + cs[L - 1]
        off = jnp.int32(0)
        for ch in range(NCH):
            pltpu.sync_copy(g_hbm.at[pl.ds(t * PPT + ch * CH, CH)], gvb)
            pltpu.sync_copy(s_hbm.at[pl.ds(t * PPT + ch * CH, CH)], svb)
            off = lax.fori_loop(0, CHV, cbody, off)
        nb = (off + BATCH - 1) // BATCH

        def gs(j, _):
            pltpu.async_copy(table_hbm.at[gsel.at[pl.ds(j * BATCH, BATCH)]],
                             rows, sem).wait()
            pltpu.async_copy(rows, accum.at[ssel3.at[j]], sem2,
                             add=True).wait()
            return 0
        lax.fori_loop(0, nb, gs, 0)
        plsc.subcore_barrier()

        pltpu.sync_copy(accum.at[pl.ds(t * rpt, rpt)],
                        sums_hbm.at[pl.ds(base + t * rpt, rpt)])


_sc_mesh = dict(core_axis_name="c", subcore_axis_name="s",
                num_cores=NC, num_subcores=NS)
_sc_params = dict(needs_layout_passes=False, use_tc_tiling_on_sc=False)


def _edge_agg(xp, vertex, edges):
    body = functools.partial(_seg_agg_body, B_CSC, B_PASSES, True)
    return pl.kernel(
        body,
        out_type=(jax.ShapeDtypeStruct((E_PAD, D), jnp.float32),
                  jax.ShapeDtypeStruct((NW * E_PAD,), jnp.float32)),
        mesh=plsc.VectorSubcoreMesh(**_sc_mesh),
        compiler_params=pltpu.CompilerParams(**_sc_params),
        scratch_types=[
            pltpu.VMEM((CH,), jnp.int32),
            pltpu.VMEM((CH,), jnp.int32),
            pltpu.VMEM((SELCAP,), jnp.int32),
            pltpu.VMEM((NBMAX, BATCH), jnp.int32),
            pltpu.VMEM((BATCH, D), jnp.float32),
            pltpu.VMEM((E_PAD,), jnp.float32),
            pltpu.VMEM_SHARED((B_CSC + 8, D), jnp.float32),
            pltpu.SemaphoreType.DMA,
            pltpu.SemaphoreType.DMA,
        ],
    )(xp, vertex, edges)


def _vertex_agg(xe, vertex, edges):
    body = functools.partial(_seg_agg_body, C_CSC, C_PASSES, False)
    return pl.kernel(
        body,
        out_type=jax.ShapeDtypeStruct((V_PAD, D), jnp.float32),
        mesh=plsc.VectorSubcoreMesh(**_sc_mesh),
        compiler_params=pltpu.CompilerParams(**_sc_params),
        scratch_types=[
            pltpu.VMEM((CH,), jnp.int32),
            pltpu.VMEM((CH,), jnp.int32),
            pltpu.VMEM((SELCAP,), jnp.int32),
            pltpu.VMEM((NBMAX, BATCH), jnp.int32),
            pltpu.VMEM((BATCH, D), jnp.float32),
            pltpu.VMEM_SHARED((C_CSC + 8, D), jnp.float32),
            pltpu.SemaphoreType.DMA,
            pltpu.SemaphoreType.DMA,
        ],
    )(xe, edges, vertex)


def _mm_body(x_ref, w_ref, o_ref):
    o_ref[...] = jnp.dot(x_ref[...], w_ref[...],
                         preferred_element_type=jnp.float32)


def _matmul(x, w):
    m, k = x.shape
    _, n = w.shape
    bm = 1000
    return pl.pallas_call(
        _mm_body,
        grid=(m // bm,),
        in_specs=[pl.BlockSpec((bm, k), lambda i: (i, 0)),
                  pl.BlockSpec((k, n), lambda i: (0, 0))],
        out_specs=pl.BlockSpec((bm, n), lambda i: (i, 0)),
        out_shape=jax.ShapeDtypeStruct((m, n), jnp.float32),
    )(x, w)


def _csum_body(c_ref, o_ref):
    o_ref[...] = jnp.sum(c_ref[...], axis=0)


def _count_combine(cnts):
    c3 = cnts.reshape(NW, E_PAD, 1)
    bm = 1024
    return pl.pallas_call(
        _csum_body,
        grid=(E_PAD // bm,),
        in_specs=[pl.BlockSpec((NW, bm, 1), lambda i: (0, i, 0))],
        out_specs=pl.BlockSpec((bm, 1), lambda i: (i, 0)),
        out_shape=jax.ShapeDtypeStruct((E_PAD, 1), jnp.float32),
    )(c3)


BM_E = 1024


def _mean_body(s_ref, c_ref, o_ref):
    o_ref[...] = s_ref[...] / jnp.maximum(c_ref[...], 1.0)


def _edge_mean(sums, cnt):
    return pl.pallas_call(
        _mean_body,
        grid=(E_PAD // BM_E,),
        in_specs=[pl.BlockSpec((BM_E, D), lambda i: (i, 0)),
                  pl.BlockSpec((BM_E, 1), lambda i: (i, 0))],
        out_specs=pl.BlockSpec((BM_E, D), lambda i: (i, 0)),
        out_shape=jax.ShapeDtypeStruct((E_PAD, D), jnp.float32),
    )(sums, cnt)


def _fin_body(xp_ref, v_ref, o_ref):
    s = xp_ref[...] + v_ref[...]
    ss = jnp.sum(s * s, axis=1, keepdims=True)
    scale = jnp.where(ss > 0, lax.rsqrt(ss), 0.0)
    o_ref[...] = s * scale


def _finalize(xp, xv):
    bm = 1000
    return pl.pallas_call(
        _fin_body,
        grid=(N_NODES // bm,),
        in_specs=[pl.BlockSpec((bm, D), lambda i: (i, 0)),
                  pl.BlockSpec((bm, D), lambda i: (i, 0))],
        out_specs=pl.BlockSpec((bm, D), lambda i: (i, 0)),
        out_shape=jax.ShapeDtypeStruct((N_NODES, D), jnp.float32),
    )(xp, xv)


def kernel(X, vertex, edges, W):
    xp = _matmul(X, W)
    sums, cnts = _edge_agg(xp, vertex, edges)
    cnt = _count_combine(cnts)
    xe = _edge_mean(sums, cnt)
    xv = _vertex_agg(xe, vertex, edges)
    return _finalize(xp, xv[:N_NODES])


# speedup vs baseline: 1.1767x; 1.1767x over previous
import functools

import jax
import jax.numpy as jnp
from jax import lax
from jax.experimental import pallas as pl
from jax.experimental.pallas import tpu as pltpu
from jax.experimental.pallas import tpu_sc as plsc

N_NODES = 10000
N_EDGES = 20000
N_PAIRS = 160000
D = 512

NC = 2
NS = 16
L = 16

NW = NC * NS
PPT = N_PAIRS // NS
CH = 2000
NCH = PPT // CH
CHV = CH // L
BATCH = 32
SELCAP = ((PPT + BATCH - 1) // BATCH) * BATCH + L
NBMAX = SELCAP // BATCH + 1
SHIFT = BATCH.bit_length() - 1

B_CSC = 2048
B_PASSES = 5
E_PAD = B_PASSES * NC * B_CSC
C_CSC = 2688
C_PASSES = 2
V_PAD = C_PASSES * NC * C_CSC


def _seg_agg_body(csc, passes, with_counts,
                  table_hbm, g_hbm, s_hbm, *rest):
    if with_counts:
        (sums_hbm, cnts_hbm, gvb0, svb0, gvb1, svb1, gsel, ssel3, rows, hist,
         accum, sem, sem2, semp0, semp1) = rest
    else:
        (sums_hbm, gvb0, svb0, gvb1, svb1, gsel, ssel3, rows, accum,
         sem, sem2, semp0, semp1) = rest
        cnts_hbm = hist = None
    gbufs = (gvb0, gvb1)
    sbufs = (svb0, svb1)
    psems = (semp0, semp1)

    c = lax.axis_index("c")
    t = lax.axis_index("s")
    rpt = csc // NS

    if with_counts:
        zv = jnp.zeros((L,), jnp.float32)
        def hzero(i, _):
            hist[pl.ds(i * L, L)] = zv
            return 0
        lax.fori_loop(0, E_PAD // L, hzero, 0)
        onev = jnp.full((L,), 1.0, jnp.float32)
        @pl.when(c == 0)
        def _():
            descs = [None, None]
            def issue(ch):
                par = ch % 2
                descs[par] = pltpu.async_copy(
                    s_hbm.at[pl.ds(t * PPT + ch * CH, CH)],
                    sbufs[par], psems[par])
            issue(0)
            for ch in range(NCH):
                par = ch % 2
                descs[par].wait()
                if ch + 1 < NCH:
                    issue(ch + 1)
                svb = sbufs[par]
                def hbody(i, _):
                    plsc.addupdate_scatter(hist, [svb[pl.ds(i * L, L)]], onev)
                    return 0
                lax.fori_loop(0, CHV, hbody, 0)
        wid = t * NC + c
        pltpu.sync_copy(hist, cnts_hbm.at[pl.ds(wid * E_PAD, E_PAD)])

    for p in range(passes):
        base = (p * NC + c) * csc

        z = jnp.zeros((L,), jnp.float32)
        def zrow(r, _):
            for k in range(D // L):
                rows[r, pl.ds(k * L, L)] = z
            return 0
        lax.fori_loop(0, BATCH, zrow, 0)
        done = 0
        while done < rpt:
            n = min(BATCH, rpt - done)
            pltpu.sync_copy(rows.at[pl.ds(0, n)],
                            accum.at[pl.ds(t * rpt + done, n)])
            done += n
        @pl.when(t == 0)
        def _():
            pltpu.sync_copy(rows.at[pl.ds(0, 8)], accum.at[pl.ds(csc, 8)])
        plsc.subcore_barrier()

        zi = jnp.zeros((L,), jnp.int32)
        def gfill(i, _):
            gsel[pl.ds(i * L, L)] = zi
            return 0
        lax.fori_loop(0, SELCAP // L, gfill, 0)
        dmp = jnp.full((L,), csc, jnp.int32)
        def sfill(j, _):
            for k in range(BATCH // L):
                ssel3[j, pl.ds(k * L, L)] = dmp
            return 0
        lax.fori_loop(0, NBMAX, sfill, 0)

        bvec = jnp.full((L,), base, jnp.int32)
        cvec = jnp.full((L,), csc, jnp.int32)
        def make_cbody(gvb, svb):
            def cbody(i, off):
                sv = svb[pl.ds(i * L, L)]
                gv = gvb[pl.ds(i * L, L)]
                lv = sv - bvec
                mask = (lv >= 0) & (lv < cvec)
                mi = mask.astype(jnp.int32)
                cs = plsc.cumsum(mi)
                pos = jnp.full((L,), off, jnp.int32) + cs - mi
                plsc.store_scatter(gsel, [pos], gv, mask=mask)
                plsc.store_scatter(
                    ssel3,
                    [lax.shift_right_logical(pos, SHIFT), pos & (BATCH - 1)],
                    lv, mask=mask)
                return off + cs[L - 1]
            return cbody
        descs = [None, None]
        def issue(ch):
            par = ch % 2
            dg = pltpu.async_copy(g_hbm.at[pl.ds(t * PPT + ch * CH, CH)],
                                  gbufs[par], psems[par])
            dsv = pltpu.async_copy(s_hbm.at[pl.ds(t * PPT + ch * CH, CH)],
                                   sbufs[par], psems[par])
            descs[par] = (dg, dsv)
        issue(0)
        off = jnp.int32(0)
        for ch in range(NCH):
            par = ch % 2
            descs[par][0].wait()
            descs[par][1].wait()
            if ch + 1 < NCH:
                issue(ch + 1)
            off = lax.fori_loop(0, CHV, make_cbody(gbufs[par], sbufs[par]), off)
        nb = (off + BATCH - 1) // BATCH

        def gs(j, _):
            pltpu.async_copy(table_hbm.at[gsel.at[pl.ds(j * BATCH, BATCH)]],
                             rows, sem).wait()
            pltpu.async_copy(rows, accum.at[ssel3.at[j]], sem2,
                             add=True).wait()
            return 0
        lax.fori_loop(0, nb, gs, 0)
        plsc.subcore_barrier()

        pltpu.sync_copy(accum.at[pl.ds(t * rpt, rpt)],
                        sums_hbm.at[pl.ds(base + t * rpt, rpt)])


_sc_mesh = dict(core_axis_name="c", subcore_axis_name="s",
                num_cores=NC, num_subcores=NS)
_sc_params = dict(needs_layout_passes=False, use_tc_tiling_on_sc=False)


def _edge_agg(xp, vertex, edges):
    body = functools.partial(_seg_agg_body, B_CSC, B_PASSES, True)
    return pl.kernel(
        body,
        out_type=(jax.ShapeDtypeStruct((E_PAD, D), jnp.float32),
                  jax.ShapeDtypeStruct((NW * E_PAD,), jnp.float32)),
        mesh=plsc.VectorSubcoreMesh(**_sc_mesh),
        compiler_params=pltpu.CompilerParams(**_sc_params),
        scratch_types=[
            pltpu.VMEM((CH,), jnp.int32),
            pltpu.VMEM((CH,), jnp.int32),
            pltpu.VMEM((CH,), jnp.int32),
            pltpu.VMEM((CH,), jnp.int32),
            pltpu.VMEM((SELCAP,), jnp.int32),
            pltpu.VMEM((NBMAX, BATCH), jnp.int32),
            pltpu.VMEM((BATCH, D), jnp.float32),
            pltpu.VMEM((E_PAD,), jnp.float32),
            pltpu.VMEM_SHARED((B_CSC + 8, D), jnp.float32),
            pltpu.SemaphoreType.DMA,
            pltpu.SemaphoreType.DMA,
            pltpu.SemaphoreType.DMA,
            pltpu.SemaphoreType.DMA,
        ],
    )(xp, vertex, edges)


def _vertex_agg(xe, vertex, edges):
    body = functools.partial(_seg_agg_body, C_CSC, C_PASSES, False)
    return pl.kernel(
        body,
        out_type=jax.ShapeDtypeStruct((V_PAD, D), jnp.float32),
        mesh=plsc.VectorSubcoreMesh(**_sc_mesh),
        compiler_params=pltpu.CompilerParams(**_sc_params),
        scratch_types=[
            pltpu.VMEM((CH,), jnp.int32),
            pltpu.VMEM((CH,), jnp.int32),
            pltpu.VMEM((CH,), jnp.int32),
            pltpu.VMEM((CH,), jnp.int32),
            pltpu.VMEM((SELCAP,), jnp.int32),
            pltpu.VMEM((NBMAX, BATCH), jnp.int32),
            pltpu.VMEM((BATCH, D), jnp.float32),
            pltpu.VMEM_SHARED((C_CSC + 8, D), jnp.float32),
            pltpu.SemaphoreType.DMA,
            pltpu.SemaphoreType.DMA,
            pltpu.SemaphoreType.DMA,
            pltpu.SemaphoreType.DMA,
        ],
    )(xe, edges, vertex)


def _mm_body(x_ref, w_ref, o_ref):
    o_ref[...] = jnp.dot(x_ref[...], w_ref[...],
                         preferred_element_type=jnp.float32)


def _matmul(x, w):
    m, k = x.shape
    _, n = w.shape
    bm = 1000
    return pl.pallas_call(
        _mm_body,
        grid=(m // bm,),
        in_specs=[pl.BlockSpec((bm, k), lambda i: (i, 0)),
                  pl.BlockSpec((k, n), lambda i: (0, 0))],
        out_specs=pl.BlockSpec((bm, n), lambda i: (i, 0)),
        out_shape=jax.ShapeDtypeStruct((m, n), jnp.float32),
    )(x, w)


def _csum_body(c_ref, o_ref):
    o_ref[...] = jnp.sum(c_ref[...], axis=0)


def _count_combine(cnts):
    c3 = cnts.reshape(NW, E_PAD, 1)
    bm = 1024
    return pl.pallas_call(
        _csum_body,
        grid=(E_PAD // bm,),
        in_specs=[pl.BlockSpec((NW, bm, 1), lambda i: (0, i, 0))],
        out_specs=pl.BlockSpec((bm, 1), lambda i: (i, 0)),
        out_shape=jax.ShapeDtypeStruct((E_PAD, 1), jnp.float32),
    )(c3)


BM_E = 1024


def _mean_body(s_ref, c_ref, o_ref):
    o_ref[...] = s_ref[...] / jnp.maximum(c_ref[...], 1.0)


def _edge_mean(sums, cnt):
    return pl.pallas_call(
        _mean_body,
        grid=(E_PAD // BM_E,),
        in_specs=[pl.BlockSpec((BM_E, D), lambda i: (i, 0)),
                  pl.BlockSpec((BM_E, 1), lambda i: (i, 0))],
        out_specs=pl.BlockSpec((BM_E, D), lambda i: (i, 0)),
        out_shape=jax.ShapeDtypeStruct((E_PAD, D), jnp.float32),
    )(sums, cnt)


def _fin_body(xp_ref, v_ref, o_ref):
    s = xp_ref[...] + v_ref[...]
    ss = jnp.sum(s * s, axis=1, keepdims=True)
    scale = jnp.where(ss > 0, lax.rsqrt(ss), 0.0)
    o_ref[...] = s * scale


def _finalize(xp, xv):
    bm = 1000
    return pl.pallas_call(
        _fin_body,
        grid=(N_NODES // bm,),
        in_specs=[pl.BlockSpec((bm, D), lambda i: (i, 0)),
                  pl.BlockSpec((bm, D), lambda i: (i, 0))],
        out_specs=pl.BlockSpec((bm, D), lambda i: (i, 0)),
        out_shape=jax.ShapeDtypeStruct((N_NODES, D), jnp.float32),
    )(xp, xv)


def kernel(X, vertex, edges, W):
    xp = _matmul(X, W)
    sums, cnts = _edge_agg(xp, vertex, edges)
    cnt = _count_combine(cnts)
    xe = _edge_mean(sums, cnt)
    xv = _vertex_agg(xe, vertex, edges)
    return _finalize(xp, xv[:N_NODES])


# speedup vs baseline: 1.5712x; 1.3353x over previous
import functools

import jax
import jax.numpy as jnp
from jax import lax
from jax.experimental import pallas as pl
from jax.experimental.pallas import tpu as pltpu
from jax.experimental.pallas import tpu_sc as plsc

N_NODES = 10000
N_EDGES = 20000
N_PAIRS = 160000
D = 512

NC = 2
NS = 16
L = 16

NW = NC * NS
PPT = N_PAIRS // NS
CH = 2000
NCH = PPT // CH
CHV = CH // L
BATCH = 32
SELCAP = ((PPT + BATCH - 1) // BATCH) * BATCH + L
NBMAX = SELCAP // BATCH + 1
SHIFT = BATCH.bit_length() - 1

B_CSC = 2048
B_PASSES = 5
E_PAD = B_PASSES * NC * B_CSC
C_CSC = 2688
C_PASSES = 2
V_PAD = C_PASSES * NC * C_CSC


def _seg_agg_body(csc, passes, with_counts,
                  table_hbm, g_hbm, s_hbm, *rest):
    if with_counts:
        (sums_hbm, gvb0, svb0, gvb1, svb1, gsel, ssel3, rows_a, rows_b,
         accum, sga, sgb, ssa, ssb, semp0, semp1) = rest
    else:
        (sums_hbm, gvb0, svb0, gvb1, svb1, gsel, ssel3, rows_a,
         accum, sga, ssa, semp0, semp1) = rest
        rows_b = sgb = ssb = None
    gbufs = (gvb0, gvb1)
    sbufs = (svb0, svb1)
    psems = (semp0, semp1)
    rows = rows_a

    c = lax.axis_index("c")
    t = lax.axis_index("s")
    rpt = csc // NS

    for p in range(passes):
        base = (p * NC + c) * csc

        z = jnp.zeros((L,), jnp.float32)
        def zrow(r, _):
            for k in range(D // L):
                rows[r, pl.ds(k * L, L)] = z
            return 0
        lax.fori_loop(0, BATCH, zrow, 0)
        done = 0
        while done < rpt:
            n = min(BATCH, rpt - done)
            pltpu.sync_copy(rows.at[pl.ds(0, n)],
                            accum.at[pl.ds(t * rpt + done, n)])
            done += n
        @pl.when(t == 0)
        def _():
            pltpu.sync_copy(rows.at[pl.ds(0, 8)], accum.at[pl.ds(csc, 8)])
        plsc.subcore_barrier()

        zi = jnp.zeros((L,), jnp.int32)
        def gfill(i, _):
            gsel[pl.ds(i * L, L)] = zi
            return 0
        lax.fori_loop(0, SELCAP // L, gfill, 0)
        dmp = jnp.full((L,), csc, jnp.int32)
        def sfill(j, _):
            for k in range(BATCH // L):
                ssel3[j, pl.ds(k * L, L)] = dmp
            return 0
        lax.fori_loop(0, NBMAX, sfill, 0)

        bvec = jnp.full((L,), base, jnp.int32)
        cvec = jnp.full((L,), csc, jnp.int32)
        def make_cbody(gvb, svb):
            def cbody(i, off):
                sv = svb[pl.ds(i * L, L)]
                gv = gvb[pl.ds(i * L, L)]
                lv = sv - bvec
                mask = (lv >= 0) & (lv < cvec)
                mi = mask.astype(jnp.int32)
                cs = plsc.cumsum(mi)
                pos = jnp.full((L,), off, jnp.int32) + cs - mi
                plsc.store_scatter(gsel, [pos], gv, mask=mask)
                plsc.store_scatter(
                    ssel3,
                    [lax.shift_right_logical(pos, SHIFT), pos & (BATCH - 1)],
                    lv, mask=mask)
                return off + cs[L - 1]
            return cbody
        descs = [None, None]
        def issue(ch):
            par = ch % 2
            dg = pltpu.async_copy(g_hbm.at[pl.ds(t * PPT + ch * CH, CH)],
                                  gbufs[par], psems[par])
            dsv = pltpu.async_copy(s_hbm.at[pl.ds(t * PPT + ch * CH, CH)],
                                   sbufs[par], psems[par])
            descs[par] = (dg, dsv)
        issue(0)
        off = jnp.int32(0)
        for ch in range(NCH):
            par = ch % 2
            descs[par][0].wait()
            descs[par][1].wait()
            if ch + 1 < NCH:
                issue(ch + 1)
            off = lax.fori_loop(0, CHV, make_cbody(gbufs[par], sbufs[par]), off)
        nb = (off + BATCH - 1) // BATCH

        if not with_counts:
            def gs(j, _):
                pltpu.async_copy(
                    table_hbm.at[gsel.at[pl.ds(j * BATCH, BATCH)]],
                    rows, sga).wait()
                pltpu.async_copy(rows, accum.at[ssel3.at[j]], ssa,
                                 add=True).wait()
                return 0
            lax.fori_loop(0, nb, gs, 0)
        else:
            rbufs = (rows_a, rows_b)
            gsems = (sga, sgb)
            ssems = (ssa, ssb)
            def wait_g(par):
                pltpu.make_async_copy(
                    table_hbm.at[gsel.at[pl.ds(0, BATCH)]],
                    rbufs[par], gsems[par]).wait()
            def wait_s(par):
                pltpu.make_async_copy(
                    rbufs[par], accum.at[ssel3.at[0]], ssems[par]).wait()
            def issue_g(j, par):
                pltpu.async_copy(
                    table_hbm.at[gsel.at[pl.ds(j * BATCH, BATCH)]],
                    rbufs[par], gsems[par])
            def issue_s(j, par):
                pltpu.async_copy(rbufs[par], accum.at[ssel3.at[j]],
                                 ssems[par], add=True)
            @pl.when(nb > 0)
            def _():
                issue_g(0, 0)
            def gs2(o, _):
                j0 = 2 * o
                j1 = j0 + 1
                j2 = j0 + 2
                @pl.when(j0 < nb)
                def _():
                    wait_g(0)
                @pl.when((j1 < nb) & (j1 > 1))
                def _():
                    wait_s(1)
                @pl.when(j1 < nb)
                def _():
                    issue_g(j1, 1)
                @pl.when(j0 < nb)
                def _():
                    issue_s(j0, 0)
                @pl.when(j1 < nb)
                def _():
                    wait_g(1)
                @pl.when(j2 < nb)
                def _():
                    wait_s(0)
                    issue_g(j2, 0)
                @pl.when(j1 < nb)
                def _():
                    issue_s(j1, 1)
                return 0
            lax.fori_loop(0, (nb + 1) // 2, gs2, 0)
            @pl.when(nb >= 1)
            def _():
                wait_s(0)
            @pl.when(nb >= 2)
            def _():
                wait_s(1)
        plsc.subcore_barrier()

        pltpu.sync_copy(accum.at[pl.ds(t * rpt, rpt)],
                        sums_hbm.at[pl.ds(base + t * rpt, rpt)])


_sc_mesh = dict(core_axis_name="c", subcore_axis_name="s",
                num_cores=NC, num_subcores=NS)
_sc_params = dict(needs_layout_passes=False, use_tc_tiling_on_sc=False)


def _edge_agg(xp, vertex, edges):
    body = functools.partial(_seg_agg_body, B_CSC, B_PASSES, True)
    return pl.kernel(
        body,
        out_type=jax.ShapeDtypeStruct((E_PAD, D), jnp.float32),
        mesh=plsc.VectorSubcoreMesh(**_sc_mesh),
        compiler_params=pltpu.CompilerParams(**_sc_params),
        scratch_types=[
            pltpu.VMEM((CH,), jnp.int32),
            pltpu.VMEM((CH,), jnp.int32),
            pltpu.VMEM((CH,), jnp.int32),
            pltpu.VMEM((CH,), jnp.int32),
            pltpu.VMEM((SELCAP,), jnp.int32),
            pltpu.VMEM((NBMAX, BATCH), jnp.int32),
            pltpu.VMEM((BATCH, D), jnp.float32),
            pltpu.VMEM((BATCH, D), jnp.float32),
            pltpu.VMEM_SHARED((B_CSC + 8, D), jnp.float32),
            pltpu.SemaphoreType.DMA,
            pltpu.SemaphoreType.DMA,
            pltpu.SemaphoreType.DMA,
            pltpu.SemaphoreType.DMA,
            pltpu.SemaphoreType.DMA,
            pltpu.SemaphoreType.DMA,
        ],
    )(xp, vertex, edges)


def _hist_body(e_hbm, cnts_hbm, svb0, svb1, hist, semp0, semp1):
    c = lax.axis_index("c")
    t = lax.axis_index("s")
    sbufs = (svb0, svb1)
    psems = (semp0, semp1)
    zv = jnp.zeros((L,), jnp.float32)
    def hzero(i, _):
        hist[pl.ds(i * L, L)] = zv
        return 0
    lax.fori_loop(0, E_PAD // L, hzero, 0)
    onev = jnp.full((L,), 1.0, jnp.float32)
    @pl.when(c == 0)
    def _():
        descs = [None, None]
        def issue(ch):
            par = ch % 2
            descs[par] = pltpu.async_copy(
                e_hbm.at[pl.ds(t * PPT + ch * CH, CH)],
                sbufs[par], psems[par])
        issue(0)
        for ch in range(NCH):
            par = ch % 2
            descs[par].wait()
            if ch + 1 < NCH:
                issue(ch + 1)
            svb = sbufs[par]
            def hbody(i, _):
                plsc.addupdate_scatter(hist, [svb[pl.ds(i * L, L)]], onev)
                return 0
            lax.fori_loop(0, CHV, hbody, 0)
    wid = t * NC + c
    pltpu.sync_copy(hist, cnts_hbm.at[pl.ds(wid * E_PAD, E_PAD)])


def _hist(edges):
    return pl.kernel(
        _hist_body,
        out_type=jax.ShapeDtypeStruct((NW * E_PAD,), jnp.float32),
        mesh=plsc.VectorSubcoreMesh(**_sc_mesh),
        compiler_params=pltpu.CompilerParams(**_sc_params),
        scratch_types=[
            pltpu.VMEM((CH,), jnp.int32),
            pltpu.VMEM((CH,), jnp.int32),
            pltpu.VMEM((E_PAD,), jnp.float32),
            pltpu.SemaphoreType.DMA,
            pltpu.SemaphoreType.DMA,
        ],
    )(edges)


def _vertex_agg(xe, vertex, edges):
    body = functools.partial(_seg_agg_body, C_CSC, C_PASSES, False)
    return pl.kernel(
        body,
        out_type=jax.ShapeDtypeStruct((V_PAD, D), jnp.float32),
        mesh=plsc.VectorSubcoreMesh(**_sc_mesh),
        compiler_params=pltpu.CompilerParams(**_sc_params),
        scratch_types=[
            pltpu.VMEM((CH,), jnp.int32),
            pltpu.VMEM((CH,), jnp.int32),
            pltpu.VMEM((CH,), jnp.int32),
            pltpu.VMEM((CH,), jnp.int32),
            pltpu.VMEM((SELCAP,), jnp.int32),
            pltpu.VMEM((NBMAX, BATCH), jnp.int32),
            pltpu.VMEM((BATCH, D), jnp.float32),
            pltpu.VMEM_SHARED((C_CSC + 8, D), jnp.float32),
            pltpu.SemaphoreType.DMA,
            pltpu.SemaphoreType.DMA,
            pltpu.SemaphoreType.DMA,
            pltpu.SemaphoreType.DMA,
        ],
    )(xe, edges, vertex)


def _mm_body(x_ref, w_ref, o_ref):
    o_ref[...] = jnp.dot(x_ref[...], w_ref[...],
                         preferred_element_type=jnp.float32)


def _matmul(x, w):
    m, k = x.shape
    _, n = w.shape
    bm = 1000
    return pl.pallas_call(
        _mm_body,
        grid=(m // bm,),
        in_specs=[pl.BlockSpec((bm, k), lambda i: (i, 0)),
                  pl.BlockSpec((k, n), lambda i: (0, 0))],
        out_specs=pl.BlockSpec((bm, n), lambda i: (i, 0)),
        out_shape=jax.ShapeDtypeStruct((m, n), jnp.float32),
    )(x, w)


def _csum_body(c_ref, o_ref):
    o_ref[...] = jnp.sum(c_ref[...], axis=0)


def _count_combine(cnts):
    c3 = cnts.reshape(NW, E_PAD, 1)
    bm = 1024
    return pl.pallas_call(
        _csum_body,
        grid=(E_PAD // bm,),
        in_specs=[pl.BlockSpec((NW, bm, 1), lambda i: (0, i, 0))],
        out_specs=pl.BlockSpec((bm, 1), lambda i: (i, 0)),
        out_shape=jax.ShapeDtypeStruct((E_PAD, 1), jnp.float32),
    )(c3)


BM_E = 1024


def _mean_body(s_ref, c_ref, o_ref):
    o_ref[...] = s_ref[...] / jnp.maximum(c_ref[...], 1.0)


def _edge_mean(sums, cnt):
    return pl.pallas_call(
        _mean_body,
        grid=(E_PAD // BM_E,),
        in_specs=[pl.BlockSpec((BM_E, D), lambda i: (i, 0)),
                  pl.BlockSpec((BM_E, 1), lambda i: (i, 0))],
        out_specs=pl.BlockSpec((BM_E, D), lambda i: (i, 0)),
        out_shape=jax.ShapeDtypeStruct((E_PAD, D), jnp.float32),
    )(sums, cnt)


def _fin_body(xp_ref, v_ref, o_ref):
    s = xp_ref[...] + v_ref[...]
    ss = jnp.sum(s * s, axis=1, keepdims=True)
    scale = jnp.where(ss > 0, lax.rsqrt(ss), 0.0)
    o_ref[...] = s * scale


def _finalize(xp, xv):
    bm = 1000
    return pl.pallas_call(
        _fin_body,
        grid=(N_NODES // bm,),
        in_specs=[pl.BlockSpec((bm, D), lambda i: (i, 0)),
                  pl.BlockSpec((bm, D), lambda i: (i, 0))],
        out_specs=pl.BlockSpec((bm, D), lambda i: (i, 0)),
        out_shape=jax.ShapeDtypeStruct((N_NODES, D), jnp.float32),
    )(xp, xv)


def kernel(X, vertex, edges, W):
    xp = _matmul(X, W)
    cnts = _hist(edges)
    sums = _edge_agg(xp, vertex, edges)
    cnt = _count_combine(cnts)
    xe = _edge_mean(sums, cnt)
    xv = _vertex_agg(xe, vertex, edges)
    return _finalize(xp, xv[:N_NODES])


# speedup vs baseline: 1.6174x; 1.0294x over previous
import functools

import jax
import jax.numpy as jnp
from jax import lax
from jax.experimental import pallas as pl
from jax.experimental.pallas import tpu as pltpu
from jax.experimental.pallas import tpu_sc as plsc

N_NODES = 10000
N_EDGES = 20000
N_PAIRS = 160000
D = 512

NC = 2
NS = 16
L = 16

NW = NC * NS
PPT = N_PAIRS // NS
CH = 2000
NCH = PPT // CH
CHV = CH // L
BATCH = 32
SELCAP = ((PPT + BATCH - 1) // BATCH) * BATCH + L
NBMAX = SELCAP // BATCH + 1
SHIFT = BATCH.bit_length() - 1

B_CSC = 2048
B_PASSES = 5
E_PAD = B_PASSES * NC * B_CSC
C_CSC = 2048
C_PASSES = 3
V_PAD = C_PASSES * NC * C_CSC


def _seg_agg_body(csc, passes, with_counts,
                  table_hbm, g_hbm, s_hbm, *rest):
    if with_counts:
        (sums_hbm, gvb0, svb0, gvb1, svb1, gsel, ssel3, rows_a, rows_b,
         accum, sga, sgb, ssa, ssb, semp0, semp1) = rest
    else:
        (sums_hbm, gvb0, svb0, gvb1, svb1, gsel, ssel3, rows_a,
         accum, sga, ssa, semp0, semp1) = rest
        rows_b = sgb = ssb = None
    gbufs = (gvb0, gvb1)
    sbufs = (svb0, svb1)
    psems = (semp0, semp1)
    rows = rows_a

    c = lax.axis_index("c")
    t = lax.axis_index("s")
    rpt = csc // NS

    for p in range(passes):
        base = (p * NC + c) * csc

        z = jnp.zeros((L,), jnp.float32)
        def zrow(r, _):
            for k in range(D // L):
                rows[r, pl.ds(k * L, L)] = z
            return 0
        lax.fori_loop(0, BATCH, zrow, 0)
        done = 0
        while done < rpt:
            n = min(BATCH, rpt - done)
            pltpu.sync_copy(rows.at[pl.ds(0, n)],
                            accum.at[pl.ds(t * rpt + done, n)])
            done += n
        @pl.when(t == 0)
        def _():
            pltpu.sync_copy(rows.at[pl.ds(0, 8)], accum.at[pl.ds(csc, 8)])
        plsc.subcore_barrier()

        zi = jnp.zeros((L,), jnp.int32)
        def gfill(i, _):
            gsel[pl.ds(i * L, L)] = zi
            return 0
        lax.fori_loop(0, SELCAP // L, gfill, 0)
        dmp = jnp.full((L,), csc, jnp.int32)
        def sfill(j, _):
            for k in range(BATCH // L):
                ssel3[j, pl.ds(k * L, L)] = dmp
            return 0
        lax.fori_loop(0, NBMAX, sfill, 0)

        bvec = jnp.full((L,), base, jnp.int32)
        cvec = jnp.full((L,), csc, jnp.int32)
        def make_cbody(gvb, svb):
            def cbody(i, off):
                sv = svb[pl.ds(i * L, L)]
                gv = gvb[pl.ds(i * L, L)]
                lv = sv - bvec
                mask = (lv >= 0) & (lv < cvec)
                mi = mask.astype(jnp.int32)
                cs = plsc.cumsum(mi)
                pos = jnp.full((L,), off, jnp.int32) + cs - mi
                plsc.store_scatter(gsel, [pos], gv, mask=mask)
                plsc.store_scatter(
                    ssel3,
                    [lax.shift_right_logical(pos, SHIFT), pos & (BATCH - 1)],
                    lv, mask=mask)
                return off + cs[L - 1]
            return cbody
        descs = [None, None]
        def issue(ch):
            par = ch % 2
            dg = pltpu.async_copy(g_hbm.at[pl.ds(t * PPT + ch * CH, CH)],
                                  gbufs[par], psems[par])
            dsv = pltpu.async_copy(s_hbm.at[pl.ds(t * PPT + ch * CH, CH)],
                                   sbufs[par], psems[par])
            descs[par] = (dg, dsv)
        issue(0)
        off = jnp.int32(0)
        for ch in range(NCH):
            par = ch % 2
            descs[par][0].wait()
            descs[par][1].wait()
            if ch + 1 < NCH:
                issue(ch + 1)
            off = lax.fori_loop(0, CHV, make_cbody(gbufs[par], sbufs[par]), off)
        nb = (off + BATCH - 1) // BATCH

        if not with_counts:
            def gs(j, _):
                pltpu.async_copy(
                    table_hbm.at[gsel.at[pl.ds(j * BATCH, BATCH)]],
                    rows, sga).wait()
                pltpu.async_copy(rows, accum.at[ssel3.at[j]], ssa,
                                 add=True).wait()
                return 0
            lax.fori_loop(0, nb, gs, 0)
        else:
            rbufs = (rows_a, rows_b)
            gsems = (sga, sgb)
            ssems = (ssa, ssb)
            def wait_g(par):
                pltpu.make_async_copy(
                    table_hbm.at[gsel.at[pl.ds(0, BATCH)]],
                    rbufs[par], gsems[par]).wait()
            def wait_s(par):
                pltpu.make_async_copy(
                    rbufs[par], accum.at[ssel3.at[0]], ssems[par]).wait()
            def issue_g(j, par):
                pltpu.async_copy(
                    table_hbm.at[gsel.at[pl.ds(j * BATCH, BATCH)]],
                    rbufs[par], gsems[par])
            def issue_s(j, par):
                pltpu.async_copy(rbufs[par], accum.at[ssel3.at[j]],
                                 ssems[par], add=True)
            @pl.when(nb > 0)
            def _():
                issue_g(0, 0)
            def gs2(o, _):
                j0 = 2 * o
                j1 = j0 + 1
                j2 = j0 + 2
                @pl.when(j0 < nb)
                def _():
                    wait_g(0)
                @pl.when((j1 < nb) & (j1 > 1))
                def _():
                    wait_s(1)
                @pl.when(j1 < nb)
                def _():
                    issue_g(j1, 1)
                @pl.when(j0 < nb)
                def _():
                    issue_s(j0, 0)
                @pl.when(j1 < nb)
                def _():
                    wait_g(1)
                @pl.when(j2 < nb)
                def _():
                    wait_s(0)
                    issue_g(j2, 0)
                @pl.when(j1 < nb)
                def _():
                    issue_s(j1, 1)
                return 0
            lax.fori_loop(0, (nb + 1) // 2, gs2, 0)
            @pl.when(nb >= 1)
            def _():
                wait_s(0)
            @pl.when(nb >= 2)
            def _():
                wait_s(1)
        plsc.subcore_barrier()

        pltpu.sync_copy(accum.at[pl.ds(t * rpt, rpt)],
                        sums_hbm.at[pl.ds(base + t * rpt, rpt)])


_sc_mesh = dict(core_axis_name="c", subcore_axis_name="s",
                num_cores=NC, num_subcores=NS)
_sc_params = dict(needs_layout_passes=False, use_tc_tiling_on_sc=False)


def _edge_agg(xp, vertex, edges):
    body = functools.partial(_seg_agg_body, B_CSC, B_PASSES, True)
    return pl.kernel(
        body,
        out_type=jax.ShapeDtypeStruct((E_PAD, D), jnp.float32),
        mesh=plsc.VectorSubcoreMesh(**_sc_mesh),
        compiler_params=pltpu.CompilerParams(**_sc_params),
        scratch_types=[
            pltpu.VMEM((CH,), jnp.int32),
            pltpu.VMEM((CH,), jnp.int32),
            pltpu.VMEM((CH,), jnp.int32),
            pltpu.VMEM((CH,), jnp.int32),
            pltpu.VMEM((SELCAP,), jnp.int32),
            pltpu.VMEM((NBMAX, BATCH), jnp.int32),
            pltpu.VMEM((BATCH, D), jnp.float32),
            pltpu.VMEM((BATCH, D), jnp.float32),
            pltpu.VMEM_SHARED((B_CSC + 8, D), jnp.float32),
            pltpu.SemaphoreType.DMA,
            pltpu.SemaphoreType.DMA,
            pltpu.SemaphoreType.DMA,
            pltpu.SemaphoreType.DMA,
            pltpu.SemaphoreType.DMA,
            pltpu.SemaphoreType.DMA,
        ],
    )(xp, vertex, edges)


def _hist_body(e_hbm, cnts_hbm, svb0, svb1, hist, semp0, semp1):
    c = lax.axis_index("c")
    t = lax.axis_index("s")
    sbufs = (svb0, svb1)
    psems = (semp0, semp1)
    zv = jnp.zeros((L,), jnp.float32)
    def hzero(i, _):
        hist[pl.ds(i * L, L)] = zv
        return 0
    lax.fori_loop(0, E_PAD // L, hzero, 0)
    onev = jnp.full((L,), 1.0, jnp.float32)
    @pl.when(c == 0)
    def _():
        descs = [None, None]
        def issue(ch):
            par = ch % 2
            descs[par] = pltpu.async_copy(
                e_hbm.at[pl.ds(t * PPT + ch * CH, CH)],
                sbufs[par], psems[par])
        issue(0)
        for ch in range(NCH):
            par = ch % 2
            descs[par].wait()
            if ch + 1 < NCH:
                issue(ch + 1)
            svb = sbufs[par]
            def hbody(i, _):
                plsc.addupdate_scatter(hist, [svb[pl.ds(i * L, L)]], onev)
                return 0
            lax.fori_loop(0, CHV, hbody, 0)
    wid = t * NC + c
    pltpu.sync_copy(hist, cnts_hbm.at[pl.ds(wid * E_PAD, E_PAD)])


def _hist(edges):
    return pl.kernel(
        _hist_body,
        out_type=jax.ShapeDtypeStruct((NW * E_PAD,), jnp.float32),
        mesh=plsc.VectorSubcoreMesh(**_sc_mesh),
        compiler_params=pltpu.CompilerParams(**_sc_params),
        scratch_types=[
            pltpu.VMEM((CH,), jnp.int32),
            pltpu.VMEM((CH,), jnp.int32),
            pltpu.VMEM((E_PAD,), jnp.float32),
            pltpu.SemaphoreType.DMA,
            pltpu.SemaphoreType.DMA,
        ],
    )(edges)


def _vertex_agg(xe, vertex, edges):
    body = functools.partial(_seg_agg_body, C_CSC, C_PASSES, True)
    return pl.kernel(
        body,
        out_type=jax.ShapeDtypeStruct((V_PAD, D), jnp.float32),
        mesh=plsc.VectorSubcoreMesh(**_sc_mesh),
        compiler_params=pltpu.CompilerParams(**_sc_params),
        scratch_types=[
            pltpu.VMEM((CH,), jnp.int32),
            pltpu.VMEM((CH,), jnp.int32),
            pltpu.VMEM((CH,), jnp.int32),
            pltpu.VMEM((CH,), jnp.int32),
            pltpu.VMEM((SELCAP,), jnp.int32),
            pltpu.VMEM((NBMAX, BATCH), jnp.int32),
            pltpu.VMEM((BATCH, D), jnp.float32),
            pltpu.VMEM((BATCH, D), jnp.float32),
            pltpu.VMEM_SHARED((C_CSC + 8, D), jnp.float32),
            pltpu.SemaphoreType.DMA,
            pltpu.SemaphoreType.DMA,
            pltpu.SemaphoreType.DMA,
            pltpu.SemaphoreType.DMA,
            pltpu.SemaphoreType.DMA,
            pltpu.SemaphoreType.DMA,
        ],
    )(xe, edges, vertex)


def _mm_body(x_ref, w_ref, o_ref):
    o_ref[...] = jnp.dot(x_ref[...], w_ref[...],
                         preferred_element_type=jnp.float32)


def _matmul(x, w):
    m, k = x.shape
    _, n = w.shape
    bm = 1000
    return pl.pallas_call(
        _mm_body,
        grid=(m // bm,),
        in_specs=[pl.BlockSpec((bm, k), lambda i: (i, 0)),
                  pl.BlockSpec((k, n), lambda i: (0, 0))],
        out_specs=pl.BlockSpec((bm, n), lambda i: (i, 0)),
        out_shape=jax.ShapeDtypeStruct((m, n), jnp.float32),
    )(x, w)


def _csum_body(c_ref, o_ref):
    o_ref[...] = jnp.sum(c_ref[...], axis=0)


def _count_combine(cnts):
    c3 = cnts.reshape(NW, E_PAD, 1)
    bm = 1024
    return pl.pallas_call(
        _csum_body,
        grid=(E_PAD // bm,),
        in_specs=[pl.BlockSpec((NW, bm, 1), lambda i: (0, i, 0))],
        out_specs=pl.BlockSpec((bm, 1), lambda i: (i, 0)),
        out_shape=jax.ShapeDtypeStruct((E_PAD, 1), jnp.float32),
    )(c3)


BM_E = 1024


def _mean_body(s_ref, c_ref, o_ref):
    o_ref[...] = s_ref[...] / jnp.maximum(c_ref[...], 1.0)


def _edge_mean(sums, cnt):
    return pl.pallas_call(
        _mean_body,
        grid=(E_PAD // BM_E,),
        in_specs=[pl.BlockSpec((BM_E, D), lambda i: (i, 0)),
                  pl.BlockSpec((BM_E, 1), lambda i: (i, 0))],
        out_specs=pl.BlockSpec((BM_E, D), lambda i: (i, 0)),
        out_shape=jax.ShapeDtypeStruct((E_PAD, D), jnp.float32),
    )(sums, cnt)


def _fin_body(xp_ref, v_ref, o_ref):
    s = xp_ref[...] + v_ref[...]
    ss = jnp.sum(s * s, axis=1, keepdims=True)
    scale = jnp.where(ss > 0, lax.rsqrt(ss), 0.0)
    o_ref[...] = s * scale


def _finalize(xp, xv):
    bm = 1000
    return pl.pallas_call(
        _fin_body,
        grid=(N_NODES // bm,),
        in_specs=[pl.BlockSpec((bm, D), lambda i: (i, 0)),
                  pl.BlockSpec((bm, D), lambda i: (i, 0))],
        out_specs=pl.BlockSpec((bm, D), lambda i: (i, 0)),
        out_shape=jax.ShapeDtypeStruct((N_NODES, D), jnp.float32),
    )(xp, xv)


def kernel(X, vertex, edges, W):
    xp = _matmul(X, W)
    cnts = _hist(edges)
    sums = _edge_agg(xp, vertex, edges)
    cnt = _count_combine(cnts)
    xe = _edge_mean(sums, cnt)
    xv = _vertex_agg(xe, vertex, edges)
    return _finalize(xp, xv[:N_NODES])


# speedup vs baseline: 1.7281x; 1.0685x over previous
import functools

import jax
import jax.numpy as jnp
from jax import lax
from jax.experimental import pallas as pl
from jax.experimental.pallas import tpu as pltpu
from jax.experimental.pallas import tpu_sc as plsc

N_NODES = 10000
N_EDGES = 20000
N_PAIRS = 160000
D = 512

NC = 2
NS = 16
L = 16

NW = NC * NS
PPT = N_PAIRS // NS
CH = 2000
NCH = PPT // CH
CHV = CH // L
BATCH = 32
SELCAP = ((PPT + BATCH - 1) // BATCH) * BATCH + L
NBMAX = SELCAP // BATCH + 1
SHIFT = BATCH.bit_length() - 1

B_CSC = 2048
B_PASSES = 5
E_PAD = B_PASSES * NC * B_CSC
C_CSC = 2048
C_PASSES = 3
V_PAD = C_PASSES * NC * C_CSC


def _seg_agg_body(csc, passes, with_counts,
                  table_hbm, g_hbm, s_hbm, *rest):
    if with_counts:
        (sums_hbm, gvb0, svb0, gvb1, svb1, gsel, ssel3, rows_a, rows_b,
         accum, sga, sgb, ssa, ssb, semp0, semp1) = rest
    else:
        (sums_hbm, gvb0, svb0, gvb1, svb1, gsel, ssel3, rows_a,
         accum, sga, ssa, semp0, semp1) = rest
        rows_b = sgb = ssb = None
    gbufs = (gvb0, gvb1)
    sbufs = (svb0, svb1)
    psems = (semp0, semp1)
    rows = rows_a

    c = lax.axis_index("c")
    t = lax.axis_index("s")
    rpt = csc // NS

    for p in range(passes):
        base = (p * NC + c) * csc

        z = jnp.zeros((L,), jnp.float32)
        def zrow(r, _):
            for k in range(D // L):
                rows[r, pl.ds(k * L, L)] = z
            return 0
        lax.fori_loop(0, BATCH, zrow, 0)
        done = 0
        while done < rpt:
            n = min(BATCH, rpt - done)
            pltpu.sync_copy(rows.at[pl.ds(0, n)],
                            accum.at[pl.ds(t * rpt + done, n)])
            done += n
        @pl.when(t == 0)
        def _():
            pltpu.sync_copy(rows.at[pl.ds(0, 8)], accum.at[pl.ds(csc, 8)])
        plsc.subcore_barrier()

        zi = jnp.zeros((L,), jnp.int32)
        def gfill(i, _):
            gsel[pl.ds(i * L, L)] = zi
            return 0
        lax.fori_loop(0, SELCAP // L, gfill, 0)
        dmp = jnp.full((L,), csc, jnp.int32)
        def sfill(j, _):
            for k in range(BATCH // L):
                ssel3[j, pl.ds(k * L, L)] = dmp
            return 0
        lax.fori_loop(0, NBMAX, sfill, 0)

        bvec = jnp.full((L,), base, jnp.int32)
        cvec = jnp.full((L,), csc, jnp.int32)
        def make_cbody(gvb, svb):
            def cbody(i, off):
                sv = svb[pl.ds(i * L, L)]
                gv = gvb[pl.ds(i * L, L)]
                lv = sv - bvec
                mask = (lv >= 0) & (lv < cvec)
                mi = mask.astype(jnp.int32)
                cs = plsc.cumsum(mi)
                pos = jnp.full((L,), off, jnp.int32) + cs - mi
                plsc.store_scatter(gsel, [pos], gv, mask=mask)
                plsc.store_scatter(
                    ssel3,
                    [lax.shift_right_logical(pos, SHIFT), pos & (BATCH - 1)],
                    lv, mask=mask)
                return off + cs[L - 1]
            return cbody
        descs = [None, None]
        def issue(ch):
            par = ch % 2
            dg = pltpu.async_copy(g_hbm.at[pl.ds(t * PPT + ch * CH, CH)],
                                  gbufs[par], psems[par])
            dsv = pltpu.async_copy(s_hbm.at[pl.ds(t * PPT + ch * CH, CH)],
                                   sbufs[par], psems[par])
            descs[par] = (dg, dsv)
        issue(0)
        off = jnp.int32(0)
        for ch in range(NCH):
            par = ch % 2
            descs[par][0].wait()
            descs[par][1].wait()
            if ch + 1 < NCH:
                issue(ch + 1)
            off = lax.fori_loop(0, CHV, make_cbody(gbufs[par], sbufs[par]), off)
        nb = (off + BATCH - 1) // BATCH

        if not with_counts:
            def gs(j, _):
                pltpu.async_copy(
                    table_hbm.at[gsel.at[pl.ds(j * BATCH, BATCH)]],
                    rows, sga).wait()
                pltpu.async_copy(rows, accum.at[ssel3.at[j]], ssa,
                                 add=True).wait()
                return 0
            lax.fori_loop(0, nb, gs, 0)
        else:
            rbufs = (rows_a, rows_b)
            gsems = (sga, sgb)
            ssems = (ssa, ssb)
            def wait_g(par):
                pltpu.make_async_copy(
                    table_hbm.at[gsel.at[pl.ds(0, BATCH)]],
                    rbufs[par], gsems[par]).wait()
            def wait_s(par):
                pltpu.make_async_copy(
                    rbufs[par], accum.at[ssel3.at[0]], ssems[par]).wait()
            def issue_g(j, par):
                pltpu.async_copy(
                    table_hbm.at[gsel.at[pl.ds(j * BATCH, BATCH)]],
                    rbufs[par], gsems[par])
            def issue_s(j, par):
                pltpu.async_copy(rbufs[par], accum.at[ssel3.at[j]],
                                 ssems[par], add=True)
            @pl.when(nb > 0)
            def _():
                issue_g(0, 0)
            def gs2(o, _):
                j0 = 2 * o
                j1 = j0 + 1
                j2 = j0 + 2
                @pl.when((j1 < nb) & (j1 > 1))
                def _():
                    wait_s(1)
                @pl.when(j1 < nb)
                def _():
                    issue_g(j1, 1)
                @pl.when(j0 < nb)
                def _():
                    wait_g(0)
                    issue_s(j0, 0)
                @pl.when(j2 < nb)
                def _():
                    wait_s(0)
                    issue_g(j2, 0)
                @pl.when(j1 < nb)
                def _():
                    wait_g(1)
                    issue_s(j1, 1)
                return 0
            lax.fori_loop(0, (nb + 1) // 2, gs2, 0)
            @pl.when(nb >= 1)
            def _():
                wait_s(0)
            @pl.when(nb >= 2)
            def _():
                wait_s(1)
        plsc.subcore_barrier()

        pltpu.sync_copy(accum.at[pl.ds(t * rpt, rpt)],
                        sums_hbm.at[pl.ds(base + t * rpt, rpt)])


_sc_mesh = dict(core_axis_name="c", subcore_axis_name="s",
                num_cores=NC, num_subcores=NS)
_sc_params = dict(needs_layout_passes=False, use_tc_tiling_on_sc=False)


def _edge_agg(xp, vertex, edges):
    body = functools.partial(_seg_agg_body, B_CSC, B_PASSES, True)
    return pl.kernel(
        body,
        out_type=jax.ShapeDtypeStruct((E_PAD, D), jnp.float32),
        mesh=plsc.VectorSubcoreMesh(**_sc_mesh),
        compiler_params=pltpu.CompilerParams(**_sc_params),
        scratch_types=[
            pltpu.VMEM((CH,), jnp.int32),
            pltpu.VMEM((CH,), jnp.int32),
            pltpu.VMEM((CH,), jnp.int32),
            pltpu.VMEM((CH,), jnp.int32),
            pltpu.VMEM((SELCAP,), jnp.int32),
            pltpu.VMEM((NBMAX, BATCH), jnp.int32),
            pltpu.VMEM((BATCH, D), jnp.float32),
            pltpu.VMEM((BATCH, D), jnp.float32),
            pltpu.VMEM_SHARED((B_CSC + 8, D), jnp.float32),
            pltpu.SemaphoreType.DMA,
            pltpu.SemaphoreType.DMA,
            pltpu.SemaphoreType.DMA,
            pltpu.SemaphoreType.DMA,
            pltpu.SemaphoreType.DMA,
            pltpu.SemaphoreType.DMA,
        ],
    )(xp, vertex, edges)


def _hist_body(e_hbm, cnts_hbm, svb0, svb1, hist, semp0, semp1):
    c = lax.axis_index("c")
    t = lax.axis_index("s")
    sbufs = (svb0, svb1)
    psems = (semp0, semp1)
    zv = jnp.zeros((L,), jnp.float32)
    def hzero(i, _):
        hist[pl.ds(i * L, L)] = zv
        return 0
    lax.fori_loop(0, E_PAD // L, hzero, 0)
    onev = jnp.full((L,), 1.0, jnp.float32)
    @pl.when(c == 0)
    def _():
        descs = [None, None]
        def issue(ch):
            par = ch % 2
            descs[par] = pltpu.async_copy(
                e_hbm.at[pl.ds(t * PPT + ch * CH, CH)],
                sbufs[par], psems[par])
        issue(0)
        for ch in range(NCH):
            par = ch % 2
            descs[par].wait()
            if ch + 1 < NCH:
                issue(ch + 1)
            svb = sbufs[par]
            def hbody(i, _):
                plsc.addupdate_scatter(hist, [svb[pl.ds(i * L, L)]], onev)
                return 0
            lax.fori_loop(0, CHV, hbody, 0)
    wid = t * NC + c
    pltpu.sync_copy(hist, cnts_hbm.at[pl.ds(wid * E_PAD, E_PAD)])


def _hist(edges):
    return pl.kernel(
        _hist_body,
        out_type=jax.ShapeDtypeStruct((NW * E_PAD,), jnp.float32),
        mesh=plsc.VectorSubcoreMesh(**_sc_mesh),
        compiler_params=pltpu.CompilerParams(**_sc_params),
        scratch_types=[
            pltpu.VMEM((CH,), jnp.int32),
            pltpu.VMEM((CH,), jnp.int32),
            pltpu.VMEM((E_PAD,), jnp.float32),
            pltpu.SemaphoreType.DMA,
            pltpu.SemaphoreType.DMA,
        ],
    )(edges)


def _vertex_agg(xe, vertex, edges):
    body = functools.partial(_seg_agg_body, C_CSC, C_PASSES, True)
    return pl.kernel(
        body,
        out_type=jax.ShapeDtypeStruct((V_PAD, D), jnp.float32),
        mesh=plsc.VectorSubcoreMesh(**_sc_mesh),
        compiler_params=pltpu.CompilerParams(**_sc_params),
        scratch_types=[
            pltpu.VMEM((CH,), jnp.int32),
            pltpu.VMEM((CH,), jnp.int32),
            pltpu.VMEM((CH,), jnp.int32),
            pltpu.VMEM((CH,), jnp.int32),
            pltpu.VMEM((SELCAP,), jnp.int32),
            pltpu.VMEM((NBMAX, BATCH), jnp.int32),
            pltpu.VMEM((BATCH, D), jnp.float32),
            pltpu.VMEM((BATCH, D), jnp.float32),
            pltpu.VMEM_SHARED((C_CSC + 8, D), jnp.float32),
            pltpu.SemaphoreType.DMA,
            pltpu.SemaphoreType.DMA,
            pltpu.SemaphoreType.DMA,
            pltpu.SemaphoreType.DMA,
            pltpu.SemaphoreType.DMA,
            pltpu.SemaphoreType.DMA,
        ],
    )(xe, edges, vertex)


def _mm_body(x_ref, w_ref, o_ref):
    o_ref[...] = jnp.dot(x_ref[...], w_ref[...],
                         preferred_element_type=jnp.float32)


def _matmul(x, w):
    m, k = x.shape
    _, n = w.shape
    bm = 1000
    return pl.pallas_call(
        _mm_body,
        grid=(m // bm,),
        in_specs=[pl.BlockSpec((bm, k), lambda i: (i, 0)),
                  pl.BlockSpec((k, n), lambda i: (0, 0))],
        out_specs=pl.BlockSpec((bm, n), lambda i: (i, 0)),
        out_shape=jax.ShapeDtypeStruct((m, n), jnp.float32),
    )(x, w)


def _csum_body(c_ref, o_ref):
    o_ref[...] = jnp.sum(c_ref[...], axis=0)


def _count_combine(cnts):
    c3 = cnts.reshape(NW, E_PAD, 1)
    bm = 1024
    return pl.pallas_call(
        _csum_body,
        grid=(E_PAD // bm,),
        in_specs=[pl.BlockSpec((NW, bm, 1), lambda i: (0, i, 0))],
        out_specs=pl.BlockSpec((bm, 1), lambda i: (i, 0)),
        out_shape=jax.ShapeDtypeStruct((E_PAD, 1), jnp.float32),
    )(c3)


BM_E = 1024


def _mean_body(s_ref, c_ref, o_ref):
    o_ref[...] = s_ref[...] / jnp.maximum(c_ref[...], 1.0)


def _edge_mean(sums, cnt):
    return pl.pallas_call(
        _mean_body,
        grid=(E_PAD // BM_E,),
        in_specs=[pl.BlockSpec((BM_E, D), lambda i: (i, 0)),
                  pl.BlockSpec((BM_E, 1), lambda i: (i, 0))],
        out_specs=pl.BlockSpec((BM_E, D), lambda i: (i, 0)),
        out_shape=jax.ShapeDtypeStruct((E_PAD, D), jnp.float32),
    )(sums, cnt)


def _fin_body(xp_ref, v_ref, o_ref):
    s = xp_ref[...] + v_ref[...]
    ss = jnp.sum(s * s, axis=1, keepdims=True)
    scale = jnp.where(ss > 0, lax.rsqrt(ss), 0.0)
    o_ref[...] = s * scale


def _finalize(xp, xv):
    bm = 1000
    return pl.pallas_call(
        _fin_body,
        grid=(N_NODES // bm,),
        in_specs=[pl.BlockSpec((bm, D), lambda i: (i, 0)),
                  pl.BlockSpec((bm, D), lambda i: (i, 0))],
        out_specs=pl.BlockSpec((bm, D), lambda i: (i, 0)),
        out_shape=jax.ShapeDtypeStruct((N_NODES, D), jnp.float32),
    )(xp, xv)


def kernel(X, vertex, edges, W):
    xp = _matmul(X, W)
    cnts = _hist(edges)
    sums = _edge_agg(xp, vertex, edges)
    cnt = _count_combine(cnts)
    xe = _edge_mean(sums, cnt)
    xv = _vertex_agg(xe, vertex, edges)
    return _finalize(xp, xv[:N_NODES])
